# baseline (speedup 1.0000x reference)
import functools

import jax
import jax.numpy as jnp
from jax import lax
from jax.experimental import pallas as pl
from jax.experimental.pallas import tpu as pltpu
from jax.experimental.pallas import tpu_sc as plsc

VOCAB = 1000000
D_MODEL = 64
BATCH = 16384
HIST = 50
HIDDEN = D_MODEL // 2

NB = 4096
Q = 253952
NQBLK = Q // NB
EMB_ROW_BLOCKS = (VOCAB + NB - 1) // NB - 1

NC = 2
NS = 16
NW = NC * NS
B_PER_W = BATCH // NW
CHUNK = 2
IDX_PER_CHUNK = CHUNK * HIST
NCHUNK = B_PER_W // CHUNK
NPAIR = NCHUNK // 2
LANES = 16
NVREG = HIDDEN // LANES
INV_HIST = 1.0 / HIST


def _proj_body(w1_ref, e0_ref, e1_ref, e2_ref, e3_ref, out_ref):
  w1 = w1_ref[...]
  parts = []
  for e_ref in (e0_ref, e1_ref, e2_ref, e3_ref):
    parts.append(
        lax.dot_general(e_ref[...], w1, (((1,), (1,)), ((), ())),
                        preferred_element_type=jnp.float32))
  out_ref[...] = jnp.concatenate(parts, axis=1)


def _project_table(emb, W1):
  def e_spec(q):
    def imap(i):
      return (jnp.minimum(i + q * NQBLK, EMB_ROW_BLOCKS), 0)
    return pl.BlockSpec((NB, D_MODEL), imap)

  return pl.pallas_call(
      _proj_body,
      grid=(NQBLK,),
      in_specs=[
          pl.BlockSpec((HIDDEN, D_MODEL), lambda i: (0, 0)),
          e_spec(0), e_spec(1), e_spec(2), e_spec(3),
      ],
      out_specs=pl.BlockSpec((NB, 4 * HIDDEN), lambda i: (i, 0)),
      out_shape=jax.ShapeDtypeStruct((Q, 4 * HIDDEN), jnp.float32),
  )(W1, emb, emb, emb, emb)


def _sc_body(tok_hbm, pq_hbm, pooled_hbm, idx_v, buf0, buf1, pooled_v,
             sem0, sem1):
  wid = lax.axis_index("s") * NC + lax.axis_index("c")
  base = wid * B_PER_W

  pltpu.sync_copy(tok_hbm.at[wid], idx_v)

  def start(ci, buf, sem):
    pltpu.make_async_copy(pq_hbm.at[idx_v.at[ci]], buf, sem).start()

  def wait(ci, buf, sem):
    pltpu.make_async_copy(pq_hbm.at[idx_v.at[ci]], buf, sem).wait()

  def reduce_chunk(buf, ci):
    for r0 in range(CHUNK):
      def rstep(r, acc):
        return tuple(acc[k] + buf[r0 * HIST + r, pl.ds(k * LANES, LANES)]
                     for k in range(NVREG))
      acc = lax.fori_loop(
          0, HIST, rstep,
          tuple(jnp.zeros((LANES,), jnp.float32) for _ in range(NVREG)),
          unroll=5)
      row = ci * CHUNK + r0
      for k in range(NVREG):
        pooled_v[row, pl.ds(k * LANES, LANES)] = acc[k] * INV_HIST

  start(0, buf0, sem0)
  start(1, buf1, sem1)

  def pair(p, _):
    ca = 2 * p
    cb = 2 * p + 1
    wait(ca, buf0, sem0)
    reduce_chunk(buf0, ca)

    @pl.when(p < NPAIR - 1)
    def _():
      start(ca + 2, buf0, sem0)

    wait(cb, buf1, sem1)
    reduce_chunk(buf1, cb)

    @pl.when(p < NPAIR - 1)
    def _():
      start(cb + 2, buf1, sem1)

    return 0

  lax.fori_loop(0, NPAIR, pair, 0)

  pltpu.sync_copy(pooled_v, pooled_hbm.at[pl.ds(base, B_PER_W)])


@functools.cache
def _sc_gather_mean():
  return pl.kernel(
      _sc_body,
      out_type=jax.ShapeDtypeStruct((BATCH, HIDDEN), jnp.float32),
      mesh=plsc.VectorSubcoreMesh(core_axis_name="c", subcore_axis_name="s",
                                  num_cores=NC, num_subcores=NS),
      compiler_params=pltpu.CompilerParams(use_tc_tiling_on_sc=False),
      scratch_types=[
          pltpu.VMEM((NCHUNK, IDX_PER_CHUNK), jnp.int32),
          pltpu.VMEM((IDX_PER_CHUNK, HIDDEN), jnp.float32),
          pltpu.VMEM((IDX_PER_CHUNK, HIDDEN), jnp.float32),
          pltpu.VMEM((B_PER_W, HIDDEN), jnp.float32),
          pltpu.SemaphoreType.DMA,
          pltpu.SemaphoreType.DMA,
      ],
  )


def _tail_body(b2_ref, h_ref, b1_ref, w2_ref, out_ref):
  h = jnp.maximum(h_ref[...] + b1_ref[...], 0.0)
  o = jnp.sum(h * w2_ref[...], axis=1, keepdims=True)
  out_ref[...] = jax.nn.sigmoid(o + b2_ref[0])


def _tail(hpre, b1, W2, b2):
  return pl.pallas_call(
      _tail_body,
      out_shape=jax.ShapeDtypeStruct((BATCH, 1), jnp.float32),
      in_specs=[
          pl.BlockSpec(memory_space=pltpu.SMEM),
          pl.BlockSpec(memory_space=pltpu.VMEM),
          pl.BlockSpec(memory_space=pltpu.VMEM),
          pl.BlockSpec(memory_space=pltpu.VMEM),
      ],
  )(b2, hpre, b1, W2)


def kernel(token_ids, emb, W1, b1, W2, b2):
  tok = token_ids.astype(jnp.int32)
  tok_sc = (4 * (tok % Q) + tok // Q).reshape(NW, NCHUNK, IDX_PER_CHUNK)
  pq = _project_table(emb, W1)
  pq_rows = pq.reshape(4 * Q, HIDDEN)
  hpre = _sc_gather_mean()(tok_sc, pq_rows)
  return _tail(hpre, b1.reshape(1, HIDDEN), W2, b2)


# speedup vs baseline: 1.7656x; 1.7656x over previous
import functools

import jax
import jax.numpy as jnp
from jax import lax
from jax.experimental import pallas as pl
from jax.experimental.pallas import tpu as pltpu
from jax.experimental.pallas import tpu_sc as plsc

VOCAB = 1000000
D_MODEL = 64
BATCH = 16384
HIST = 50
HIDDEN = D_MODEL // 2

NB = 4096
Q = 253952
NQBLK = Q // NB
EMB_ROW_BLOCKS = (VOCAB + NB - 1) // NB - 1

NC = 2
NS = 16
NW = NC * NS
B_PER_W = BATCH // NW
CHUNK = 2
IDX_PER_CHUNK = CHUNK * HIST
NCHUNK = B_PER_W // CHUNK
NPAIR = NCHUNK // 2
LANES = 16
NVREG = HIDDEN // LANES
INV_HIST = 1.0 / HIST


def _proj_body(w1_ref, e0_ref, e1_ref, e2_ref, e3_ref, out_ref):
  w1 = w1_ref[...]
  parts = []
  for e_ref in (e0_ref, e1_ref, e2_ref, e3_ref):
    parts.append(
        lax.dot_general(w1, e_ref[...], (((1,), (0,)), ((), ())),
                        preferred_element_type=jnp.float32))
  out_ref[...] = jnp.concatenate(parts, axis=0).T


def _project_table(emb_t, W1):
  def e_spec(q):
    def imap(i):
      return (0, jnp.minimum(i + q * NQBLK, EMB_ROW_BLOCKS))
    return pl.BlockSpec((D_MODEL, NB), imap)

  return pl.pallas_call(
      _proj_body,
      grid=(NQBLK,),
      in_specs=[
          pl.BlockSpec((HIDDEN, D_MODEL), lambda i: (0, 0)),
          e_spec(0), e_spec(1), e_spec(2), e_spec(3),
      ],
      out_specs=pl.BlockSpec((NB, 4 * HIDDEN), lambda i: (i, 0)),
      out_shape=jax.ShapeDtypeStruct((Q, 4 * HIDDEN), jnp.float32),
  )(W1, emb_t, emb_t, emb_t, emb_t)


def _sc_body(tok_hbm, pq_hbm, pooled_hbm, idx_v, buf0, buf1, pooled_v,
             sem0, sem1):
  wid = lax.axis_index("s") * NC + lax.axis_index("c")
  base = wid * B_PER_W

  pltpu.sync_copy(tok_hbm.at[wid], idx_v)

  def start(ci, buf, sem):
    pltpu.make_async_copy(pq_hbm.at[idx_v.at[ci]], buf, sem).start()

  def wait(ci, buf, sem):
    pltpu.make_async_copy(pq_hbm.at[idx_v.at[ci]], buf, sem).wait()

  def reduce_chunk(buf, ci):
    for r0 in range(CHUNK):
      def rstep(r, acc):
        return tuple(acc[k] + buf[r0 * HIST + r, pl.ds(k * LANES, LANES)]
                     for k in range(NVREG))
      acc = lax.fori_loop(
          0, HIST, rstep,
          tuple(jnp.zeros((LANES,), jnp.float32) for _ in range(NVREG)),
          unroll=5)
      row = ci * CHUNK + r0
      for k in range(NVREG):
        pooled_v[row, pl.ds(k * LANES, LANES)] = acc[k] * INV_HIST

  start(0, buf0, sem0)
  start(1, buf1, sem1)

  def pair(p, _):
    ca = 2 * p
    cb = 2 * p + 1
    wait(ca, buf0, sem0)
    reduce_chunk(buf0, ca)

    @pl.when(p < NPAIR - 1)
    def _():
      start(ca + 2, buf0, sem0)

    wait(cb, buf1, sem1)
    reduce_chunk(buf1, cb)

    @pl.when(p < NPAIR - 1)
    def _():
      start(cb + 2, buf1, sem1)

    return 0

  lax.fori_loop(0, NPAIR, pair, 0)

  pltpu.sync_copy(pooled_v, pooled_hbm.at[pl.ds(base, B_PER_W)])


@functools.cache
def _sc_gather_mean():
  return pl.kernel(
      _sc_body,
      out_type=jax.ShapeDtypeStruct((BATCH, HIDDEN), jnp.float32),
      mesh=plsc.VectorSubcoreMesh(core_axis_name="c", subcore_axis_name="s",
                                  num_cores=NC, num_subcores=NS),
      compiler_params=pltpu.CompilerParams(use_tc_tiling_on_sc=False),
      scratch_types=[
          pltpu.VMEM((NCHUNK, IDX_PER_CHUNK), jnp.int32),
          pltpu.VMEM((IDX_PER_CHUNK, HIDDEN), jnp.float32),
          pltpu.VMEM((IDX_PER_CHUNK, HIDDEN), jnp.float32),
          pltpu.VMEM((B_PER_W, HIDDEN), jnp.float32),
          pltpu.SemaphoreType.DMA,
          pltpu.SemaphoreType.DMA,
      ],
  )


def _tail_body(b2_ref, h_ref, b1_ref, w2_ref, out_ref):
  h = jnp.maximum(h_ref[...] + b1_ref[...], 0.0)
  o = jnp.sum(h * w2_ref[...], axis=1, keepdims=True)
  out_ref[...] = jax.nn.sigmoid(o + b2_ref[0])


def _tail(hpre, b1, W2, b2):
  return pl.pallas_call(
      _tail_body,
      out_shape=jax.ShapeDtypeStruct((BATCH, 1), jnp.float32),
      in_specs=[
          pl.BlockSpec(memory_space=pltpu.SMEM),
          pl.BlockSpec(memory_space=pltpu.VMEM),
          pl.BlockSpec(memory_space=pltpu.VMEM),
          pl.BlockSpec(memory_space=pltpu.VMEM),
      ],
  )(b2, hpre, b1, W2)


def kernel(token_ids, emb, W1, b1, W2, b2):
  tok = token_ids.astype(jnp.int32)
  tok_sc = (4 * (tok % Q) + tok // Q).reshape(NW, NCHUNK, IDX_PER_CHUNK)
  pq = _project_table(emb.T.astype(jnp.bfloat16),
                      W1.astype(jnp.bfloat16))
  pq_rows = pq.reshape(4 * Q, HIDDEN)
  hpre = _sc_gather_mean()(tok_sc, pq_rows)
  return _tail(hpre, b1.reshape(1, HIDDEN), W2, b2)


# speedup vs baseline: 2.2811x; 1.2919x over previous
import functools

import jax
import jax.numpy as jnp
from jax import lax
from jax.experimental import pallas as pl
from jax.experimental.pallas import tpu as pltpu
from jax.experimental.pallas import tpu_sc as plsc

VOCAB = 1000000
D_MODEL = 64
BATCH = 16384
HIST = 50
HIDDEN = D_MODEL // 2

NB = 4096
Q = 253952
NQBLK = Q // NB
EMB_ROW_BLOCKS = (VOCAB + NB - 1) // NB - 1

NC = 2
NS = 16
NW = NC * NS
B_PER_W = BATCH // NW
CHUNK = 2
IDX_PER_CHUNK = CHUNK * HIST
NCHUNK = B_PER_W // CHUNK
NPAIR = NCHUNK // 2
LANES = 16
NVREG = HIDDEN // LANES
INV_HIST = 1.0 / HIST


def _proj_body(w1_ref, e0_ref, e1_ref, e2_ref, e3_ref, out_ref):
  w1 = w1_ref[...]
  parts = []
  for e_ref in (e0_ref, e1_ref, e2_ref, e3_ref):
    parts.append(
        lax.dot_general(w1, e_ref[...], (((1,), (0,)), ((), ())),
                        preferred_element_type=jnp.float32))
  out_ref[...] = jnp.concatenate(parts, axis=0).T


def _project_table(emb_t, W1):
  def e_spec(q):
    def imap(i):
      return (0, jnp.minimum(i + q * NQBLK, EMB_ROW_BLOCKS))
    return pl.BlockSpec((D_MODEL, NB), imap)

  return pl.pallas_call(
      _proj_body,
      grid=(NQBLK,),
      in_specs=[
          pl.BlockSpec((HIDDEN, D_MODEL), lambda i: (0, 0)),
          e_spec(0), e_spec(1), e_spec(2), e_spec(3),
      ],
      out_specs=pl.BlockSpec((NB, 4 * HIDDEN), lambda i: (i, 0)),
      out_shape=jax.ShapeDtypeStruct((Q, 4 * HIDDEN), jnp.float32),
  )(W1, emb_t, emb_t, emb_t, emb_t)


def _sc_body(tok_hbm, pq_hbm, pooled_hbm, idx_v, buf0, buf1, pooled_v,
             sem0, sem1):
  wid = lax.axis_index("s") * NC + lax.axis_index("c")
  base = wid * B_PER_W

  pltpu.sync_copy(tok_hbm.at[wid], idx_v)

  def start(ci, buf, sem):
    pltpu.make_async_copy(pq_hbm.at[idx_v.at[ci]], buf, sem).start()

  def wait(ci, buf, sem):
    pltpu.make_async_copy(pq_hbm.at[idx_v.at[ci]], buf, sem).wait()

  def reduce_chunk(buf, ci):
    accs = [[buf[r0 * HIST, pl.ds(k * LANES, LANES)] for k in range(NVREG)]
            for r0 in range(CHUNK)]
    for r in range(1, HIST):
      for r0 in range(CHUNK):
        for k in range(NVREG):
          accs[r0][k] = accs[r0][k] + buf[r0 * HIST + r,
                                          pl.ds(k * LANES, LANES)]
    for r0 in range(CHUNK):
      row = ci * CHUNK + r0
      for k in range(NVREG):
        pooled_v[row, pl.ds(k * LANES, LANES)] = accs[r0][k] * INV_HIST

  start(0, buf0, sem0)
  start(1, buf1, sem1)

  def pair(p, _):
    ca = 2 * p
    cb = 2 * p + 1
    wait(ca, buf0, sem0)
    reduce_chunk(buf0, ca)

    @pl.when(p < NPAIR - 1)
    def _():
      start(ca + 2, buf0, sem0)

    wait(cb, buf1, sem1)
    reduce_chunk(buf1, cb)

    @pl.when(p < NPAIR - 1)
    def _():
      start(cb + 2, buf1, sem1)

    return 0

  lax.fori_loop(0, NPAIR, pair, 0)

  pltpu.sync_copy(pooled_v, pooled_hbm.at[pl.ds(base, B_PER_W)])


@functools.cache
def _sc_gather_mean():
  return pl.kernel(
      _sc_body,
      out_type=jax.ShapeDtypeStruct((BATCH, HIDDEN), jnp.float32),
      mesh=plsc.VectorSubcoreMesh(core_axis_name="c", subcore_axis_name="s",
                                  num_cores=NC, num_subcores=NS),
      compiler_params=pltpu.CompilerParams(use_tc_tiling_on_sc=False),
      scratch_types=[
          pltpu.VMEM((NCHUNK, IDX_PER_CHUNK), jnp.int32),
          pltpu.VMEM((IDX_PER_CHUNK, HIDDEN), jnp.float32),
          pltpu.VMEM((IDX_PER_CHUNK, HIDDEN), jnp.float32),
          pltpu.VMEM((B_PER_W, HIDDEN), jnp.float32),
          pltpu.SemaphoreType.DMA,
          pltpu.SemaphoreType.DMA,
      ],
  )


def _tail_body(b2_ref, h_ref, b1_ref, w2_ref, out_ref):
  h = jnp.maximum(h_ref[...] + b1_ref[...], 0.0)
  o = jnp.sum(h * w2_ref[...], axis=1, keepdims=True)
  out_ref[...] = jax.nn.sigmoid(o + b2_ref[0])


def _tail(hpre, b1, W2, b2):
  return pl.pallas_call(
      _tail_body,
      out_shape=jax.ShapeDtypeStruct((BATCH, 1), jnp.float32),
      in_specs=[
          pl.BlockSpec(memory_space=pltpu.SMEM),
          pl.BlockSpec(memory_space=pltpu.VMEM),
          pl.BlockSpec(memory_space=pltpu.VMEM),
          pl.BlockSpec(memory_space=pltpu.VMEM),
      ],
  )(b2, hpre, b1, W2)


def kernel(token_ids, emb, W1, b1, W2, b2):
  tok = token_ids.astype(jnp.int32)
  tok_sc = (4 * (tok % Q) + tok // Q).reshape(NW, NCHUNK, IDX_PER_CHUNK)
  pq = _project_table(emb.T, W1)
  pq_rows = pq.reshape(4 * Q, HIDDEN)
  hpre = _sc_gather_mean()(tok_sc, pq_rows)
  return _tail(hpre, b1.reshape(1, HIDDEN), W2, b2)


# speedup vs baseline: 2.6263x; 1.1513x over previous
import functools

import jax
import jax.numpy as jnp
from jax import lax
from jax.experimental import pallas as pl
from jax.experimental.pallas import tpu as pltpu
from jax.experimental.pallas import tpu_sc as plsc

VOCAB = 1000000
D_MODEL = 64
BATCH = 16384
HIST = 50
HIDDEN = D_MODEL // 2

NB = 4096
NCHNK = 8
QC = 126976
NQBLK = QC // NB
EMB_ROW_BLOCKS = (VOCAB + NB - 1) // NB - 1

NC = 2
NS = 16
NW = NC * NS
B_PER_W = BATCH // NW
CHUNK = 2
IDX_PER_CHUNK = CHUNK * HIST
NCHUNK = B_PER_W // CHUNK
NPAIR = NCHUNK // 2
LANES = 16
NVREG = HIDDEN // LANES
INV_HIST = 1.0 / HIST


def _proj_body(w1_ref, *refs):
  e_refs = refs[:NCHNK]
  out_ref = refs[NCHNK]
  w1 = w1_ref[...]
  packed = []
  for e_ref in e_refs:
    p = lax.dot_general(w1, e_ref[...], (((1,), (0,)), ((), ())),
                        preferred_element_type=jnp.float32)
    u = lax.bitcast_convert_type(p, jnp.int32)
    r = u + 32768
    hi = r[:LANES, :] & (-65536)
    lo = lax.shift_right_logical(r[LANES:, :], 16)
    packed.append(hi | lo)
  out_ref[...] = jnp.concatenate(packed, axis=0).T


def _project_table(emb_t, W1):
  def e_spec(q):
    def imap(i):
      return (0, jnp.minimum(i + q * NQBLK, EMB_ROW_BLOCKS))
    return pl.BlockSpec((D_MODEL, NB), imap)

  return pl.pallas_call(
      _proj_body,
      grid=(NQBLK,),
      in_specs=[
          pl.BlockSpec((HIDDEN, D_MODEL), lambda i: (0, 0)),
      ] + [e_spec(q) for q in range(NCHNK)],
      out_specs=pl.BlockSpec((NB, NCHNK * LANES), lambda i: (i, 0)),
      out_shape=jax.ShapeDtypeStruct((QC, NCHNK * LANES), jnp.int32),
  )(W1, *([emb_t] * NCHNK))


def _sc_body(tok_hbm, pq_hbm, pooled_hbm, idx_v, buf0, buf1, pooled_v,
             sem0, sem1):
  wid = lax.axis_index("s") * NC + lax.axis_index("c")
  base = wid * B_PER_W

  pltpu.sync_copy(tok_hbm.at[wid], idx_v)

  def start(ci, buf, sem):
    pltpu.make_async_copy(pq_hbm.at[idx_v.at[ci]], buf, sem).start()

  def wait(ci, buf, sem):
    pltpu.make_async_copy(pq_hbm.at[idx_v.at[ci]], buf, sem).wait()

  def reduce_chunk(buf, ci):
    for r0 in range(CHUNK):
      def rstep(r, acc):
        v = buf[r0 * HIST + r, pl.ds(0, LANES)]
        hi = lax.bitcast_convert_type(v & (-65536), jnp.float32)
        lo = lax.bitcast_convert_type(lax.shift_left(v, 16), jnp.float32)
        return (acc[0] + hi, acc[1] + lo)
      acc = lax.fori_loop(
          0, HIST, rstep,
          (jnp.zeros((LANES,), jnp.float32),
           jnp.zeros((LANES,), jnp.float32)),
          unroll=5)
      row = ci * CHUNK + r0
      pooled_v[row, pl.ds(0, LANES)] = acc[0] * INV_HIST
      pooled_v[row, pl.ds(LANES, LANES)] = acc[1] * INV_HIST

  start(0, buf0, sem0)
  start(1, buf1, sem1)

  def pair(p, _):
    ca = 2 * p
    cb = 2 * p + 1
    wait(ca, buf0, sem0)
    reduce_chunk(buf0, ca)

    @pl.when(p < NPAIR - 1)
    def _():
      start(ca + 2, buf0, sem0)

    wait(cb, buf1, sem1)
    reduce_chunk(buf1, cb)

    @pl.when(p < NPAIR - 1)
    def _():
      start(cb + 2, buf1, sem1)

    return 0

  lax.fori_loop(0, NPAIR, pair, 0)

  pltpu.sync_copy(pooled_v, pooled_hbm.at[pl.ds(base, B_PER_W)])


@functools.cache
def _sc_gather_mean():
  return pl.kernel(
      _sc_body,
      out_type=jax.ShapeDtypeStruct((BATCH, HIDDEN), jnp.float32),
      mesh=plsc.VectorSubcoreMesh(core_axis_name="c", subcore_axis_name="s",
                                  num_cores=NC, num_subcores=NS),
      compiler_params=pltpu.CompilerParams(use_tc_tiling_on_sc=False),
      scratch_types=[
          pltpu.VMEM((NCHUNK, IDX_PER_CHUNK), jnp.int32),
          pltpu.VMEM((IDX_PER_CHUNK, LANES), jnp.int32),
          pltpu.VMEM((IDX_PER_CHUNK, LANES), jnp.int32),
          pltpu.VMEM((B_PER_W, HIDDEN), jnp.float32),
          pltpu.SemaphoreType.DMA,
          pltpu.SemaphoreType.DMA,
      ],
  )


def _tail_body(b2_ref, h_ref, b1_ref, w2_ref, out_ref):
  h = jnp.maximum(h_ref[...] + b1_ref[...], 0.0)
  o = jnp.sum(h * w2_ref[...], axis=1, keepdims=True)
  out_ref[...] = jax.nn.sigmoid(o + b2_ref[0])


def _tail(hpre, b1, W2, b2):
  return pl.pallas_call(
      _tail_body,
      out_shape=jax.ShapeDtypeStruct((BATCH, 1), jnp.float32),
      in_specs=[
          pl.BlockSpec(memory_space=pltpu.SMEM),
          pl.BlockSpec(memory_space=pltpu.VMEM),
          pl.BlockSpec(memory_space=pltpu.VMEM),
          pl.BlockSpec(memory_space=pltpu.VMEM),
      ],
  )(b2, hpre, b1, W2)


def kernel(token_ids, emb, W1, b1, W2, b2):
  tok = token_ids.astype(jnp.int32)
  tok_sc = (NCHNK * (tok % QC) + tok // QC).reshape(NW, NCHUNK, IDX_PER_CHUNK)
  pq = _project_table(emb.T, W1)
  pq_rows = pq.reshape(NCHNK * QC, LANES)
  hpre = _sc_gather_mean()(tok_sc, pq_rows)
  return _tail(hpre, b1.reshape(1, HIDDEN), W2, b2)


# speedup vs baseline: 3.2321x; 1.2307x over previous
import functools

import jax
import jax.numpy as jnp
from jax import lax
from jax.experimental import pallas as pl
from jax.experimental.pallas import tpu as pltpu
from jax.experimental.pallas import tpu_sc as plsc

VOCAB = 1000000
D_MODEL = 64
BATCH = 16384
HIST = 50
HIDDEN = D_MODEL // 2

NB = 4096
NCHNK = 8
QC = 126976
NQBLK = QC // NB
EMB_ROW_BLOCKS = (VOCAB + NB - 1) // NB - 1

NC = 2
NS = 16
NW = NC * NS
B_PER_W = BATCH // NW
CHUNK = 2
IDX_PER_CHUNK = CHUNK * HIST
NCHUNK = B_PER_W // CHUNK
NBUF = 4
NGRP = NCHUNK // NBUF
LANES = 16
NVREG = HIDDEN // LANES
INV_HIST = 1.0 / HIST


def _proj_body(w1_ref, *refs):
  e_refs = refs[:NCHNK]
  out_ref = refs[NCHNK]
  w1 = w1_ref[...]
  packed = []
  for e_ref in e_refs:
    p = lax.dot_general(w1, e_ref[...], (((1,), (0,)), ((), ())),
                        preferred_element_type=jnp.float32)
    u = lax.bitcast_convert_type(p, jnp.int32)
    r = u + 32768
    hi = r[:LANES, :] & (-65536)
    lo = lax.shift_right_logical(r[LANES:, :], 16)
    packed.append(hi | lo)
  out_ref[...] = jnp.concatenate(packed, axis=0).T


def _project_table(emb_t, W1):
  def e_spec(q):
    def imap(i):
      return (0, jnp.minimum(i + q * NQBLK, EMB_ROW_BLOCKS))
    return pl.BlockSpec((D_MODEL, NB), imap)

  return pl.pallas_call(
      _proj_body,
      grid=(NQBLK,),
      in_specs=[
          pl.BlockSpec((HIDDEN, D_MODEL), lambda i: (0, 0)),
      ] + [e_spec(q) for q in range(NCHNK)],
      out_specs=pl.BlockSpec((NB, NCHNK * LANES), lambda i: (i, 0)),
      out_shape=jax.ShapeDtypeStruct((QC, NCHNK * LANES), jnp.int32),
  )(W1, *([emb_t] * NCHNK))


def _sc_body(tok_hbm, pq_hbm, pooled_hbm, idx_v, buf0, buf1, buf2, buf3,
             pooled_v, sem0, sem1, sem2, sem3):
  bufs = (buf0, buf1, buf2, buf3)
  sems = (sem0, sem1, sem2, sem3)
  wid = lax.axis_index("s") * NC + lax.axis_index("c")
  base = wid * B_PER_W

  pltpu.sync_copy(tok_hbm.at[wid], idx_v)

  def start(ci, buf, sem):
    pltpu.make_async_copy(pq_hbm.at[idx_v.at[ci]], buf, sem).start()

  def wait(ci, buf, sem):
    pltpu.make_async_copy(pq_hbm.at[idx_v.at[ci]], buf, sem).wait()

  def reduce_chunk(buf, ci):
    for r0 in range(CHUNK):
      def rstep(r, acc):
        v = buf[r0 * HIST + r, pl.ds(0, LANES)]
        hi = lax.bitcast_convert_type(v & (-65536), jnp.float32)
        lo = lax.bitcast_convert_type(lax.shift_left(v, 16), jnp.float32)
        return (acc[0] + hi, acc[1] + lo)
      acc = lax.fori_loop(
          0, HIST, rstep,
          (jnp.zeros((LANES,), jnp.float32),
           jnp.zeros((LANES,), jnp.float32)),
          unroll=5)
      row = ci * CHUNK + r0
      pooled_v[row, pl.ds(0, LANES)] = acc[0] * INV_HIST
      pooled_v[row, pl.ds(LANES, LANES)] = acc[1] * INV_HIST

  for j in range(NBUF):
    start(j, bufs[j], sems[j])

  def grp(g, _):
    for j in range(NBUF):
      ci = g * NBUF + j
      wait(ci, bufs[j], sems[j])
      reduce_chunk(bufs[j], ci)

      @pl.when(g < NGRP - 1)
      def _():
        start(ci + NBUF, bufs[j], sems[j])

    return 0

  lax.fori_loop(0, NGRP, grp, 0)

  pltpu.sync_copy(pooled_v, pooled_hbm.at[pl.ds(base, B_PER_W)])


@functools.cache
def _sc_gather_mean():
  return pl.kernel(
      _sc_body,
      out_type=jax.ShapeDtypeStruct((BATCH, HIDDEN), jnp.float32),
      mesh=plsc.VectorSubcoreMesh(core_axis_name="c", subcore_axis_name="s",
                                  num_cores=NC, num_subcores=NS),
      compiler_params=pltpu.CompilerParams(use_tc_tiling_on_sc=False),
      scratch_types=[
          pltpu.VMEM((NCHUNK, IDX_PER_CHUNK), jnp.int32),
          pltpu.VMEM((IDX_PER_CHUNK, LANES), jnp.int32),
          pltpu.VMEM((IDX_PER_CHUNK, LANES), jnp.int32),
          pltpu.VMEM((IDX_PER_CHUNK, LANES), jnp.int32),
          pltpu.VMEM((IDX_PER_CHUNK, LANES), jnp.int32),
          pltpu.VMEM((B_PER_W, HIDDEN), jnp.float32),
          pltpu.SemaphoreType.DMA,
          pltpu.SemaphoreType.DMA,
          pltpu.SemaphoreType.DMA,
          pltpu.SemaphoreType.DMA,
      ],
  )


def _tail_body(b2_ref, h_ref, b1_ref, w2_ref, out_ref):
  h = jnp.maximum(h_ref[...] + b1_ref[...], 0.0)
  o = jnp.sum(h * w2_ref[...], axis=1, keepdims=True)
  out_ref[...] = jax.nn.sigmoid(o + b2_ref[0])


def _tail(hpre, b1, W2, b2):
  return pl.pallas_call(
      _tail_body,
      out_shape=jax.ShapeDtypeStruct((BATCH, 1), jnp.float32),
      in_specs=[
          pl.BlockSpec(memory_space=pltpu.SMEM),
          pl.BlockSpec(memory_space=pltpu.VMEM),
          pl.BlockSpec(memory_space=pltpu.VMEM),
          pl.BlockSpec(memory_space=pltpu.VMEM),
      ],
  )(b2, hpre, b1, W2)


def kernel(token_ids, emb, W1, b1, W2, b2):
  tok = token_ids.astype(jnp.int32)
  tok_sc = (NCHNK * (tok % QC) + tok // QC).reshape(NW, NCHUNK, IDX_PER_CHUNK)
  pq = _project_table(emb.T, W1)
  pq_rows = pq.reshape(NCHNK * QC, LANES)
  hpre = _sc_gather_mean()(tok_sc, pq_rows)
  return _tail(hpre, b1.reshape(1, HIDDEN), W2, b2)


# speedup vs baseline: 3.6251x; 1.1216x over previous
import functools

import jax
import jax.numpy as jnp
from jax import lax
from jax.experimental import pallas as pl
from jax.experimental.pallas import tpu as pltpu
from jax.experimental.pallas import tpu_sc as plsc

VOCAB = 1000000
D_MODEL = 64
BATCH = 16384
HIST = 50
HIDDEN = D_MODEL // 2

NB = 4096
NCHNK = 8
QC = 126976
NQBLK = QC // NB
EMB_ROW_BLOCKS = (VOCAB + NB - 1) // NB - 1

NC = 2
NS = 16
NW = NC * NS
B_PER_W = BATCH // NW
CHUNK = 2
IDX_PER_CHUNK = CHUNK * HIST
NCHUNK = B_PER_W // CHUNK
NBUF = 8
NGRP = NCHUNK // NBUF
LANES = 16
NVREG = HIDDEN // LANES
INV_HIST = 1.0 / HIST


def _proj_body(w1_ref, *refs):
  e_refs = refs[:NCHNK]
  out_ref = refs[NCHNK]
  w1 = w1_ref[...]
  packed = []
  for e_ref in e_refs:
    p = lax.dot_general(w1, e_ref[...], (((1,), (0,)), ((), ())),
                        preferred_element_type=jnp.float32)
    u = lax.bitcast_convert_type(p, jnp.int32)
    r = u + 32768
    hi = r[:LANES, :] & (-65536)
    lo = lax.shift_right_logical(r[LANES:, :], 16)
    packed.append(hi | lo)
  out_ref[...] = jnp.concatenate(packed, axis=0).T


def _project_table(emb_t, W1):
  def e_spec(q):
    def imap(i):
      return (0, jnp.minimum(i + q * NQBLK, EMB_ROW_BLOCKS))
    return pl.BlockSpec((D_MODEL, NB), imap)

  return pl.pallas_call(
      _proj_body,
      grid=(NQBLK,),
      in_specs=[
          pl.BlockSpec((HIDDEN, D_MODEL), lambda i: (0, 0)),
      ] + [e_spec(q) for q in range(NCHNK)],
      out_specs=pl.BlockSpec((NB, NCHNK * LANES), lambda i: (i, 0)),
      out_shape=jax.ShapeDtypeStruct((QC, NCHNK * LANES), jnp.int32),
  )(W1, *([emb_t] * NCHNK))


def _sc_body(tok_hbm, pq_hbm, pooled_hbm, idx_v, *rest):
  bufs = rest[:NBUF]
  pooled_v = rest[NBUF]
  sems = rest[NBUF + 1:]
  wid = lax.axis_index("s") * NC + lax.axis_index("c")
  base = wid * B_PER_W

  pltpu.sync_copy(tok_hbm.at[wid], idx_v)

  def start(ci, buf, sem):
    pltpu.make_async_copy(pq_hbm.at[idx_v.at[ci]], buf, sem).start()

  def wait(ci, buf, sem):
    pltpu.make_async_copy(pq_hbm.at[idx_v.at[ci]], buf, sem).wait()

  def reduce_chunk(buf, ci):
    for r0 in range(CHUNK):
      def rstep(r, acc):
        v = buf[r0 * HIST + r, pl.ds(0, LANES)]
        hi = lax.bitcast_convert_type(v & (-65536), jnp.float32)
        lo = lax.bitcast_convert_type(lax.shift_left(v, 16), jnp.float32)
        return (acc[0] + hi, acc[1] + lo)
      acc = lax.fori_loop(
          0, HIST, rstep,
          (jnp.zeros((LANES,), jnp.float32),
           jnp.zeros((LANES,), jnp.float32)),
          unroll=5)
      row = ci * CHUNK + r0
      pooled_v[row, pl.ds(0, LANES)] = acc[0] * INV_HIST
      pooled_v[row, pl.ds(LANES, LANES)] = acc[1] * INV_HIST

  for j in range(NBUF):
    start(j, bufs[j], sems[j])

  def grp(g, _):
    for j in range(NBUF):
      ci = g * NBUF + j
      wait(ci, bufs[j], sems[j])
      reduce_chunk(bufs[j], ci)

      @pl.when(g < NGRP - 1)
      def _():
        start(ci + NBUF, bufs[j], sems[j])

    return 0

  lax.fori_loop(0, NGRP, grp, 0)

  pltpu.sync_copy(pooled_v, pooled_hbm.at[pl.ds(base, B_PER_W)])


@functools.cache
def _sc_gather_mean():
  return pl.kernel(
      _sc_body,
      out_type=jax.ShapeDtypeStruct((BATCH, HIDDEN), jnp.float32),
      mesh=plsc.VectorSubcoreMesh(core_axis_name="c", subcore_axis_name="s",
                                  num_cores=NC, num_subcores=NS),
      compiler_params=pltpu.CompilerParams(use_tc_tiling_on_sc=False),
      scratch_types=[
          pltpu.VMEM((NCHUNK, IDX_PER_CHUNK), jnp.int32),
      ] + [pltpu.VMEM((IDX_PER_CHUNK, LANES), jnp.int32)
           for _ in range(NBUF)] + [
          pltpu.VMEM((B_PER_W, HIDDEN), jnp.float32),
      ] + [pltpu.SemaphoreType.DMA for _ in range(NBUF)],
  )


def _tail_body(b2_ref, h_ref, b1_ref, w2_ref, out_ref):
  h = jnp.maximum(h_ref[...] + b1_ref[...], 0.0)
  o = jnp.sum(h * w2_ref[...], axis=1, keepdims=True)
  out_ref[...] = jax.nn.sigmoid(o + b2_ref[0])


def _tail(hpre, b1, W2, b2):
  return pl.pallas_call(
      _tail_body,
      out_shape=jax.ShapeDtypeStruct((BATCH, 1), jnp.float32),
      in_specs=[
          pl.BlockSpec(memory_space=pltpu.SMEM),
          pl.BlockSpec(memory_space=pltpu.VMEM),
          pl.BlockSpec(memory_space=pltpu.VMEM),
          pl.BlockSpec(memory_space=pltpu.VMEM),
      ],
  )(b2, hpre, b1, W2)


def kernel(token_ids, emb, W1, b1, W2, b2):
  tok = token_ids.astype(jnp.int32)
  tok_sc = (NCHNK * (tok % QC) + tok // QC).reshape(NW, NCHUNK, IDX_PER_CHUNK)
  pq = _project_table(emb.T, W1)
  pq_rows = pq.reshape(NCHNK * QC, LANES)
  hpre = _sc_gather_mean()(tok_sc, pq_rows)
  return _tail(hpre, b1.reshape(1, HIDDEN), W2, b2)
